# baseline (speedup 1.0000x reference)
import jax
import jax.numpy as jnp
from jax import lax
from jax.experimental import pallas as pl
from jax.experimental.pallas import tpu as pltpu
from jax.experimental.pallas import tpu_sc as plsc

N = 10000
E = 320000
NUM_TYPES = 512
EMB = 128
HID = 128
NUM_GRAPHS = 64

NC = 2
NS = 16
NW = NC * NS
EB = 128
NB_E = 80
CB = 40
EP = NW * NB_E * EB
NP = 10240
RPT = NP // NS
XB = 40
NB_X = (NP // NW) // XB

_f32 = jnp.float32
_bf16 = jnp.bfloat16


def _worker_id():
  c = lax.axis_index("c")
  s = lax.axis_index("s")
  return s * NC + c, c, s


def _sc_gather_body(xt_hbm, emb_hbm, x_out, xt_idx, rows_v, sem):
  wid, c, s = _worker_id()
  pltpu.sync_copy(xt_hbm.at[wid], xt_idx)

  def gather_body(j):
    pltpu.async_copy(emb_hbm.at[xt_idx.at[j]], rows_v, sem).wait()
    pltpu.sync_copy(rows_v,
                    x_out.at[pl.ds(wid * (NB_X * XB) + j * XB, XB)])
  pl.loop(0, NB_X)(gather_body)


def _sc_conv_body(g_hbm, src_hbm, dst_hbm, zeros_hbm,
                  acc_out,
                  src_c, dst_c, rows0, rows1, sem0, sem1, acc):
  wid, c, s = _worker_id()
  pltpu.sync_copy(zeros_hbm.at[pl.ds(s * RPT, RPT)],
                  acc.at[pl.ds(s * RPT, RPT)])
  plsc.subcore_barrier()

  def chunk(j):
    base = wid * NB_E + j * CB
    pltpu.sync_copy(src_hbm.at[pl.ds(base, CB)], src_c)
    pltpu.sync_copy(dst_hbm.at[pl.ds(base, CB)], dst_c)
    pltpu.async_copy(g_hbm.at[src_c.at[0]], rows0, sem0)

    def pair(t):
      pltpu.make_async_copy(g_hbm.at[src_c.at[t]], rows0, sem0).wait()
      pltpu.async_copy(g_hbm.at[src_c.at[t + 1]], rows1, sem1)
      pltpu.sync_copy(rows0, acc.at[dst_c.at[t]], add=True)
      pltpu.make_async_copy(g_hbm.at[src_c.at[t + 1]], rows1, sem1).wait()

      @pl.when(t + 2 < CB)
      def _():
        pltpu.async_copy(g_hbm.at[src_c.at[t + 2]], rows0, sem0)
      pltpu.sync_copy(rows1, acc.at[dst_c.at[t + 1]], add=True)

    pl.loop(0, CB, step=2)(pair)

  pl.loop(0, NB_E // CB)(chunk)

  plsc.subcore_barrier()
  pltpu.sync_copy(acc.at[pl.ds(s * RPT, RPT)],
                  acc_out.at[c, pl.ds(s * RPT, RPT)])


def _make_sc_kernels():
  mesh = plsc.VectorSubcoreMesh(core_axis_name="c", subcore_axis_name="s")
  gather = pl.kernel(
      _sc_gather_body,
      out_type=jax.ShapeDtypeStruct((NP, EMB), _f32),
      mesh=mesh,
      scratch_types=[
          pltpu.VMEM((NB_X, XB), jnp.int32),
          pltpu.VMEM((XB, EMB), _f32),
          pltpu.SemaphoreType.DMA,
      ],
      name="gnn_sc_gather",
  )
  conv = pl.kernel(
      _sc_conv_body,
      out_type=jax.ShapeDtypeStruct((NC, NP, HID), _f32),
      mesh=mesh,
      scratch_types=[
          pltpu.VMEM((CB, EB), jnp.int32),
          pltpu.VMEM((CB, EB), jnp.int32),
          pltpu.VMEM((EB, HID), _f32),
          pltpu.VMEM((EB, HID), _f32),
          pltpu.SemaphoreType.DMA,
          pltpu.SemaphoreType.DMA,
          pltpu.VMEM_SHARED((NP, HID), _f32),
      ],
      name="gnn_sc_conv",
  )
  return gather, conv


_ROWS_B = 1024
_GRID = NP // _ROWS_B
_EC = 1024
_EGRID = EP // _EC
_HI = NP // 128


def _tc_hist_body(dr_ref, dc_ref, dm_ref):
  pid = pl.program_id(0)
  hi_r = dr_ref[0] >> 7
  lo_c = dc_ref[0] & 127
  mask_hi = (lax.broadcasted_iota(jnp.int32, (_HI, _EC), 0)
             == jnp.broadcast_to(hi_r, (_HI, _EC))).astype(_bf16)
  onehot = (lax.broadcasted_iota(jnp.int32, (_EC, 128), 1)
            == jnp.broadcast_to(lo_c, (_EC, 128))).astype(_bf16)
  contrib = jnp.dot(mask_hi, onehot, preferred_element_type=_f32)

  @pl.when(pid == 0)
  def _():
    dm_ref[...] = contrib

  @pl.when(pid > 0)
  def _():
    dm_ref[...] = dm_ref[...] + contrib


def _tc_prep_body(x_ref, df_ref, w1_ref, g1_ref, dinv_ref):
  pid = pl.program_id(0)
  dinv = lax.rsqrt(df_ref[...] + 1.0)
  dinvb = jnp.broadcast_to(dinv, (_ROWS_B, HID))
  h = jnp.dot(x_ref[...], w1_ref[...], preferred_element_type=_f32)
  row = pid * _ROWS_B + lax.broadcasted_iota(jnp.int32, (_ROWS_B, HID), 0)
  g1_ref[...] = jnp.where(row < N, h * dinvb, 0.0)
  dinv_ref[...] = dinvb


def _tc_mid_body(a_ref, g1_ref, dinv_ref, b1_ref, w2_ref, g2_ref):
  pid = pl.program_id(0)
  dinv = dinv_ref[...]
  z1 = dinv * (a_ref[0] + a_ref[1] + g1_ref[...]) + b1_ref[...]
  z1 = jnp.maximum(z1, 0.0)
  h2 = jnp.dot(z1, w2_ref[...], preferred_element_type=_f32)
  row = pid * _ROWS_B + lax.broadcasted_iota(jnp.int32, (_ROWS_B, HID), 0)
  g2_ref[...] = jnp.where(row < N, h2 * dinv, 0.0)


def _tc_pool_body(a_ref, g2_ref, dinv_ref, b2_ref, bt_ref, out_ref):
  pid = pl.program_id(0)
  z2 = dinv_ref[...] * (a_ref[0] + a_ref[1] + g2_ref[...]) + b2_ref[...]
  bt = bt_ref[0]
  gid = lax.broadcasted_iota(jnp.int32, (NUM_GRAPHS, _ROWS_B), 0)
  onehot = (gid == jnp.broadcast_to(bt, (NUM_GRAPHS, _ROWS_B))).astype(_f32)
  contrib = jnp.dot(onehot, z2, preferred_element_type=_f32)

  @pl.when(pid == 0)
  def _():
    out_ref[...] = contrib

  @pl.when(pid > 0)
  def _():
    out_ref[...] = out_ref[...] + contrib


def _row_spec(width):
  return pl.BlockSpec((_ROWS_B, width), lambda i: (i, 0))


def _acc_spec():
  return pl.BlockSpec((NC, _ROWS_B, HID), lambda i: (0, i, 0))


def _const_spec(shape):
  nd = len(shape)
  return pl.BlockSpec(shape, lambda i: (0,) * nd)


def _tc_hist(dst_r, dst_c):
  return pl.pallas_call(
      _tc_hist_body,
      grid=(_EGRID,),
      in_specs=[pl.BlockSpec((1, 1, _EC), lambda i: (i, 0, 0)),
                pl.BlockSpec((1, _EC, 1), lambda i: (i, 0, 0))],
      out_specs=_const_spec((_HI, 128)),
      out_shape=jax.ShapeDtypeStruct((_HI, 128), _f32),
  )(dst_r, dst_c)


def _tc_prep(x, deg_flat, w1):
  return pl.pallas_call(
      _tc_prep_body,
      grid=(_GRID,),
      in_specs=[_row_spec(EMB), _row_spec(1),
                _const_spec((EMB, HID))],
      out_specs=[_row_spec(HID), _row_spec(HID)],
      out_shape=[jax.ShapeDtypeStruct((NP, HID), _f32),
                 jax.ShapeDtypeStruct((NP, HID), _f32)],
  )(x, deg_flat, w1)


def _tc_mid(a, g1, dinv, b1, w2):
  return pl.pallas_call(
      _tc_mid_body,
      grid=(_GRID,),
      in_specs=[_acc_spec(), _row_spec(HID), _row_spec(HID),
                _const_spec((1, HID)), _const_spec((HID, HID))],
      out_specs=_row_spec(HID),
      out_shape=jax.ShapeDtypeStruct((NP, HID), _f32),
  )(a, g1, dinv, b1, w2)


def _tc_pool(a, g2, dinv, b2, batch3):
  return pl.pallas_call(
      _tc_pool_body,
      grid=(_GRID,),
      in_specs=[_acc_spec(), _row_spec(HID), _row_spec(HID),
                _const_spec((1, HID)),
                pl.BlockSpec((1, 1, _ROWS_B), lambda i: (i, 0, 0))],
      out_specs=_const_spec((NUM_GRAPHS, HID)),
      out_shape=jax.ShapeDtypeStruct((NUM_GRAPHS, HID), _f32),
  )(a, g2, dinv, b2, batch3)


@jax.jit
def kernel(x_type, edge_index, batch, emb, W1, b1, W2, b2):
  i32 = jnp.int32
  src = edge_index[0].astype(i32)
  dst = edge_index[1].astype(i32)

  pad = N + (jnp.arange(EP - E, dtype=i32) % (NP - N))
  srcp = jnp.concatenate([src, pad])
  dstp = jnp.concatenate([dst, pad])
  src2 = srcp.reshape(NW * NB_E, EB)
  dst2 = dstp.reshape(NW * NB_E, EB)
  dst_r = dstp.reshape(_EGRID, 1, _EC)
  dst_c = dstp.reshape(_EGRID, _EC, 1)
  xt = jnp.concatenate(
      [x_type.astype(i32), jnp.zeros((NP - N,), i32)]).reshape(NW, NB_X, XB)
  batch3 = jnp.concatenate(
      [batch.astype(i32),
       jnp.full((NP - N,), NUM_GRAPHS, i32)]).reshape(_GRID, 1, _ROWS_B)

  zeros128 = jnp.zeros((NP, HID), _f32)

  sc_gather, conv = _make_sc_kernels()

  deg_mat = _tc_hist(dst_r, dst_c)
  x = sc_gather(xt, emb)
  g1, dinv = _tc_prep(x, deg_mat.reshape(NP, 1), W1)

  acc1 = conv(g1, src2, dst2, zeros128)
  g2 = _tc_mid(acc1, g1, dinv, b1.reshape(1, HID), W2)

  acc2 = conv(g2, src2, dst2, zeros128)
  out = _tc_pool(acc2, g2, dinv, b2.reshape(1, HID), batch3)
  return out


# speedup vs baseline: 1.2759x; 1.2759x over previous
import jax
import jax.numpy as jnp
from jax import lax
from jax.experimental import pallas as pl
from jax.experimental.pallas import tpu as pltpu
from jax.experimental.pallas import tpu_sc as plsc

N = 10000
E = 320000
NUM_TYPES = 512
EMB = 128
HID = 128
NUM_GRAPHS = 64

NC = 2
NS = 16
NW = NC * NS
EB = 128
NB_E = 80
CB = 40
EP = NW * NB_E * EB
NP = 10240
RPT = NP // NS
XB = 40
NB_X = (NP // NW) // XB

_f32 = jnp.float32
_bf16 = jnp.bfloat16


def _worker_id():
  c = lax.axis_index("c")
  s = lax.axis_index("s")
  return s * NC + c, c, s


def _sc_gather_body(xt_hbm, emb_hbm, x_out, xt_idx, rows_v, sem):
  wid, c, s = _worker_id()
  pltpu.sync_copy(xt_hbm.at[wid], xt_idx)

  def gather_body(j):
    pltpu.async_copy(emb_hbm.at[xt_idx.at[j]], rows_v, sem).wait()
    pltpu.sync_copy(rows_v,
                    x_out.at[pl.ds(wid * (NB_X * XB) + j * XB, XB)])
  pl.loop(0, NB_X)(gather_body)


def _sc_conv_body(g_hbm, src_hbm, dst_hbm, zeros_hbm,
                  acc_out,
                  src_c, dst_c, rows0, rows1, sem0, sem1, acc):
  wid, c, s = _worker_id()
  pltpu.sync_copy(zeros_hbm.at[pl.ds(s * RPT, RPT)],
                  acc.at[pl.ds(s * RPT, RPT)])
  plsc.subcore_barrier()

  def chunk(j):
    base = wid * NB_E + j * CB
    pltpu.sync_copy(src_hbm.at[pl.ds(base, CB)], src_c)
    pltpu.sync_copy(dst_hbm.at[pl.ds(base, CB)], dst_c)
    pltpu.async_copy(g_hbm.at[src_c.at[0]], rows0, sem0)

    def pair(t):
      pltpu.make_async_copy(g_hbm.at[src_c.at[t]], rows0, sem0).wait()
      pltpu.async_copy(g_hbm.at[src_c.at[t + 1]], rows1, sem1)
      pltpu.sync_copy(rows0, acc.at[dst_c.at[t]], add=True)
      pltpu.make_async_copy(g_hbm.at[src_c.at[t + 1]], rows1, sem1).wait()

      @pl.when(t + 2 < CB)
      def _():
        pltpu.async_copy(g_hbm.at[src_c.at[t + 2]], rows0, sem0)
      pltpu.sync_copy(rows1, acc.at[dst_c.at[t + 1]], add=True)

    pl.loop(0, CB, step=2)(pair)

  pl.loop(0, NB_E // CB)(chunk)

  plsc.subcore_barrier()
  pltpu.sync_copy(acc.at[pl.ds(s * RPT, RPT)],
                  acc_out.at[c, pl.ds(s * RPT, RPT)])


def _make_sc_kernels():
  mesh = plsc.VectorSubcoreMesh(core_axis_name="c", subcore_axis_name="s")
  gather = pl.kernel(
      _sc_gather_body,
      out_type=jax.ShapeDtypeStruct((NP, EMB), _f32),
      mesh=mesh,
      scratch_types=[
          pltpu.VMEM((NB_X, XB), jnp.int32),
          pltpu.VMEM((XB, EMB), _f32),
          pltpu.SemaphoreType.DMA,
      ],
      name="gnn_sc_gather",
  )
  conv = pl.kernel(
      _sc_conv_body,
      out_type=jax.ShapeDtypeStruct((NC, NP, HID), _f32),
      mesh=mesh,
      scratch_types=[
          pltpu.VMEM((CB, EB), jnp.int32),
          pltpu.VMEM((CB, EB), jnp.int32),
          pltpu.VMEM((EB, HID), _f32),
          pltpu.VMEM((EB, HID), _f32),
          pltpu.SemaphoreType.DMA,
          pltpu.SemaphoreType.DMA,
          pltpu.VMEM_SHARED((NP, HID), _f32),
      ],
      name="gnn_sc_conv",
  )
  return gather, conv


_ROWS_B = 1024
_GRID = NP // _ROWS_B
_EC = 8192
_EGRID = EP // _EC
_HI = NP // 128


def _tc_hist_body(dr_ref, dc_ref, dm_ref):
  pid = pl.program_id(0)
  hi_r = dr_ref[0] >> 7
  lo_c = dc_ref[0] & 127
  mask_hi = (lax.broadcasted_iota(jnp.int32, (_HI, _EC), 0)
             == jnp.broadcast_to(hi_r, (_HI, _EC))).astype(_bf16)
  onehot = (lax.broadcasted_iota(jnp.int32, (_EC, 128), 1)
            == jnp.broadcast_to(lo_c, (_EC, 128))).astype(_bf16)
  contrib = jnp.dot(mask_hi, onehot, preferred_element_type=_f32)

  @pl.when(pid == 0)
  def _():
    dm_ref[...] = contrib

  @pl.when(pid > 0)
  def _():
    dm_ref[...] = dm_ref[...] + contrib


def _tc_prep_body(x_ref, df_ref, w1_ref, g1_ref, dinv_ref):
  pid = pl.program_id(0)
  dinv = lax.rsqrt(df_ref[...] + 1.0)
  dinvb = jnp.broadcast_to(dinv, (_ROWS_B, HID))
  h = jnp.dot(x_ref[...], w1_ref[...], preferred_element_type=_f32)
  row = pid * _ROWS_B + lax.broadcasted_iota(jnp.int32, (_ROWS_B, HID), 0)
  g1_ref[...] = jnp.where(row < N, h * dinvb, 0.0)
  dinv_ref[...] = dinvb


def _tc_mid_body(a_ref, g1_ref, dinv_ref, b1_ref, w2_ref, g2_ref):
  pid = pl.program_id(0)
  dinv = dinv_ref[...]
  z1 = dinv * (a_ref[0] + a_ref[1] + g1_ref[...]) + b1_ref[...]
  z1 = jnp.maximum(z1, 0.0)
  h2 = jnp.dot(z1, w2_ref[...], preferred_element_type=_f32)
  row = pid * _ROWS_B + lax.broadcasted_iota(jnp.int32, (_ROWS_B, HID), 0)
  g2_ref[...] = jnp.where(row < N, h2 * dinv, 0.0)


def _tc_pool_body(a_ref, g2_ref, dinv_ref, b2_ref, bt_ref, out_ref):
  pid = pl.program_id(0)
  z2 = dinv_ref[...] * (a_ref[0] + a_ref[1] + g2_ref[...]) + b2_ref[...]
  bt = bt_ref[0]
  gid = lax.broadcasted_iota(jnp.int32, (NUM_GRAPHS, _ROWS_B), 0)
  onehot = (gid == jnp.broadcast_to(bt, (NUM_GRAPHS, _ROWS_B))).astype(_f32)
  contrib = jnp.dot(onehot, z2, preferred_element_type=_f32)

  @pl.when(pid == 0)
  def _():
    out_ref[...] = contrib

  @pl.when(pid > 0)
  def _():
    out_ref[...] = out_ref[...] + contrib


def _row_spec(width):
  return pl.BlockSpec((_ROWS_B, width), lambda i: (i, 0))


def _acc_spec():
  return pl.BlockSpec((NC, _ROWS_B, HID), lambda i: (0, i, 0))


def _const_spec(shape):
  nd = len(shape)
  return pl.BlockSpec(shape, lambda i: (0,) * nd)


def _tc_hist(dst_r, dst_c):
  return pl.pallas_call(
      _tc_hist_body,
      grid=(_EGRID,),
      in_specs=[pl.BlockSpec((1, 1, _EC), lambda i: (i, 0, 0)),
                pl.BlockSpec((1, _EC, 1), lambda i: (i, 0, 0))],
      out_specs=_const_spec((_HI, 128)),
      out_shape=jax.ShapeDtypeStruct((_HI, 128), _f32),
  )(dst_r, dst_c)


def _tc_prep(x, deg_flat, w1):
  return pl.pallas_call(
      _tc_prep_body,
      grid=(_GRID,),
      in_specs=[_row_spec(EMB), _row_spec(1),
                _const_spec((EMB, HID))],
      out_specs=[_row_spec(HID), _row_spec(HID)],
      out_shape=[jax.ShapeDtypeStruct((NP, HID), _f32),
                 jax.ShapeDtypeStruct((NP, HID), _f32)],
  )(x, deg_flat, w1)


def _tc_mid(a, g1, dinv, b1, w2):
  return pl.pallas_call(
      _tc_mid_body,
      grid=(_GRID,),
      in_specs=[_acc_spec(), _row_spec(HID), _row_spec(HID),
                _const_spec((1, HID)), _const_spec((HID, HID))],
      out_specs=_row_spec(HID),
      out_shape=jax.ShapeDtypeStruct((NP, HID), _f32),
  )(a, g1, dinv, b1, w2)


def _tc_pool(a, g2, dinv, b2, batch3):
  return pl.pallas_call(
      _tc_pool_body,
      grid=(_GRID,),
      in_specs=[_acc_spec(), _row_spec(HID), _row_spec(HID),
                _const_spec((1, HID)),
                pl.BlockSpec((1, 1, _ROWS_B), lambda i: (i, 0, 0))],
      out_specs=_const_spec((NUM_GRAPHS, HID)),
      out_shape=jax.ShapeDtypeStruct((NUM_GRAPHS, HID), _f32),
  )(a, g2, dinv, b2, batch3)


@jax.jit
def kernel(x_type, edge_index, batch, emb, W1, b1, W2, b2):
  i32 = jnp.int32
  src = edge_index[0].astype(i32)
  dst = edge_index[1].astype(i32)

  pad = N + (jnp.arange(EP - E, dtype=i32) % (NP - N))
  srcp = jnp.concatenate([src, pad])
  dstp = jnp.concatenate([dst, pad])
  src2 = srcp.reshape(NW * NB_E, EB)
  dst2 = dstp.reshape(NW * NB_E, EB)
  dst_r = dstp.reshape(_EGRID, 1, _EC)
  dst_c = dstp.reshape(_EGRID, _EC, 1)
  xt = jnp.concatenate(
      [x_type.astype(i32), jnp.zeros((NP - N,), i32)]).reshape(NW, NB_X, XB)
  batch3 = jnp.concatenate(
      [batch.astype(i32),
       jnp.full((NP - N,), NUM_GRAPHS, i32)]).reshape(_GRID, 1, _ROWS_B)

  zeros128 = jnp.zeros((NP, HID), _f32)

  sc_gather, conv = _make_sc_kernels()

  deg_mat = _tc_hist(dst_r, dst_c)
  x = sc_gather(xt, emb)
  g1, dinv = _tc_prep(x, deg_mat.reshape(NP, 1), W1)

  acc1 = conv(g1, src2, dst2, zeros128)
  g2 = _tc_mid(acc1, g1, dinv, b1.reshape(1, HID), W2)

  acc2 = conv(g2, src2, dst2, zeros128)
  out = _tc_pool(acc2, g2, dinv, b2.reshape(1, HID), batch3)
  return out


# speedup vs baseline: 1.5774x; 1.2363x over previous
import jax
import jax.numpy as jnp
from jax import lax
from jax.experimental import pallas as pl
from jax.experimental.pallas import tpu as pltpu
from jax.experimental.pallas import tpu_sc as plsc

N = 10000
E = 320000
NUM_TYPES = 512
EMB = 128
HID = 128
NUM_GRAPHS = 64

NC = 2
NS = 16
NW = NC * NS
EB = 128
NB_E = 80
CB = 40
EP = NW * NB_E * EB
NP = 10240
RPT = NP // NS
XB = 40
NB_X = (NP // NW) // XB

_f32 = jnp.float32
_bf16 = jnp.bfloat16


def _worker_id():
  c = lax.axis_index("c")
  s = lax.axis_index("s")
  return s * NC + c, c, s


def _sc_deg_gather_body(xt_hbm, dst_hbm, emb_hbm, ones_hbm, zeros_hbm,
                        deg_out, x_out,
                        dst_idx, xt_idx, ones_v, xr0, xr1, sx0, sx1, acc):
  wid, c, s = _worker_id()
  pltpu.sync_copy(dst_hbm.at[pl.ds(wid * NB_E, NB_E)], dst_idx)
  pltpu.sync_copy(xt_hbm.at[wid], xt_idx)
  pltpu.sync_copy(ones_hbm, ones_v)
  pltpu.sync_copy(zeros_hbm.at[pl.ds(s * RPT, RPT)],
                  acc.at[pl.ds(s * RPT, RPT)])
  plsc.subcore_barrier()

  def gx(j, buf, sem):
    pltpu.async_copy(emb_hbm.at[xt_idx.at[j]], buf, sem)

  def wx(j, buf, sem):
    pltpu.make_async_copy(emb_hbm.at[xt_idx.at[j]], buf, sem).wait()

  def deg_burst(base):
    def deg_body(b):
      pltpu.sync_copy(ones_v, acc.at[dst_idx.at[b]], add=True)
    pl.loop(base, base + NB_E // NB_X)(deg_body)

  gx(0, xr0, sx0)

  def phase(j):
    burst = NB_E // NB_X
    wx(j, xr0, sx0)
    gx(j + 1, xr1, sx1)
    deg_burst(j * burst)
    pltpu.sync_copy(xr0, x_out.at[pl.ds(wid * (NB_X * XB) + j * XB, XB)])
    wx(j + 1, xr1, sx1)
    gx(j + 2, xr0, sx0)
    deg_burst((j + 1) * burst)
    pltpu.sync_copy(xr1, x_out.at[pl.ds(wid * (NB_X * XB) + (j + 1) * XB, XB)])

  pl.loop(0, NB_X, step=2)(phase)
  wx(NB_X, xr0, sx0)

  plsc.subcore_barrier()
  pltpu.sync_copy(acc.at[pl.ds(s * RPT, RPT)],
                  deg_out.at[c, pl.ds(s * RPT, RPT)])


def _sc_conv_body(g_hbm, src_hbm, dst_hbm, zeros_hbm,
                  acc_out,
                  src_c, dst_c, rows0, rows1, sem0, sem1, acc):
  wid, c, s = _worker_id()
  pltpu.sync_copy(zeros_hbm.at[pl.ds(s * RPT, RPT)],
                  acc.at[pl.ds(s * RPT, RPT)])
  plsc.subcore_barrier()

  def chunk(j):
    base = wid * NB_E + j * CB
    pltpu.sync_copy(src_hbm.at[pl.ds(base, CB)], src_c)
    pltpu.sync_copy(dst_hbm.at[pl.ds(base, CB)], dst_c)
    pltpu.async_copy(g_hbm.at[src_c.at[0]], rows0, sem0)

    def pair(t):
      pltpu.make_async_copy(g_hbm.at[src_c.at[t]], rows0, sem0).wait()
      pltpu.async_copy(g_hbm.at[src_c.at[t + 1]], rows1, sem1)
      pltpu.sync_copy(rows0, acc.at[dst_c.at[t]], add=True)
      pltpu.make_async_copy(g_hbm.at[src_c.at[t + 1]], rows1, sem1).wait()

      @pl.when(t + 2 < CB)
      def _():
        pltpu.async_copy(g_hbm.at[src_c.at[t + 2]], rows0, sem0)
      pltpu.sync_copy(rows1, acc.at[dst_c.at[t + 1]], add=True)

    pl.loop(0, CB, step=2)(pair)

  pl.loop(0, NB_E // CB)(chunk)

  plsc.subcore_barrier()
  pltpu.sync_copy(acc.at[pl.ds(s * RPT, RPT)],
                  acc_out.at[c, pl.ds(s * RPT, RPT)])


def _make_sc_kernels():
  mesh = plsc.VectorSubcoreMesh(core_axis_name="c", subcore_axis_name="s")
  gather = pl.kernel(
      _sc_deg_gather_body,
      out_type=(
          jax.ShapeDtypeStruct((NC, NP, HID), _f32),
          jax.ShapeDtypeStruct((NP, EMB), _f32),
      ),
      mesh=mesh,
      scratch_types=[
          pltpu.VMEM((NB_E, EB), jnp.int32),
          pltpu.VMEM((NB_X + 1, XB), jnp.int32),
          pltpu.VMEM((EB, HID), _f32),
          pltpu.VMEM((XB, EMB), _f32),
          pltpu.VMEM((XB, EMB), _f32),
          pltpu.SemaphoreType.DMA,
          pltpu.SemaphoreType.DMA,
          pltpu.VMEM_SHARED((NP, HID), _f32),
      ],
      name="gnn_sc_deg_gather",
  )
  conv = pl.kernel(
      _sc_conv_body,
      out_type=jax.ShapeDtypeStruct((NC, NP, HID), _f32),
      mesh=mesh,
      scratch_types=[
          pltpu.VMEM((CB, EB), jnp.int32),
          pltpu.VMEM((CB, EB), jnp.int32),
          pltpu.VMEM((EB, HID), _f32),
          pltpu.VMEM((EB, HID), _f32),
          pltpu.SemaphoreType.DMA,
          pltpu.SemaphoreType.DMA,
          pltpu.VMEM_SHARED((NP, HID), _f32),
      ],
      name="gnn_sc_conv",
  )
  return gather, conv


_ROWS_B = 1024
_GRID = NP // _ROWS_B
_EC = 8192
_EGRID = EP // _EC
_HI = NP // 128


def _tc_hist_body(dr_ref, dc_ref, dm_ref):
  pid = pl.program_id(0)
  hi_r = dr_ref[0] >> 7
  lo_c = dc_ref[0] & 127
  mask_hi = (lax.broadcasted_iota(jnp.int32, (_HI, _EC), 0)
             == jnp.broadcast_to(hi_r, (_HI, _EC))).astype(_bf16)
  onehot = (lax.broadcasted_iota(jnp.int32, (_EC, 128), 1)
            == jnp.broadcast_to(lo_c, (_EC, 128))).astype(_bf16)
  contrib = jnp.dot(mask_hi, onehot, preferred_element_type=_f32)

  @pl.when(pid == 0)
  def _():
    dm_ref[...] = contrib

  @pl.when(pid > 0)
  def _():
    dm_ref[...] = dm_ref[...] + contrib


def _tc_prep_body(x_ref, d0_ref, d1_ref, w1_ref, g1_ref, dinv_ref):
  pid = pl.program_id(0)
  deg = d0_ref[:, 0:1] + d1_ref[:, 0:1]
  dinv = lax.rsqrt(deg + 1.0)
  dinvb = jnp.broadcast_to(dinv, (_ROWS_B, HID))
  h = jnp.dot(x_ref[...], w1_ref[...], preferred_element_type=_f32)
  row = pid * _ROWS_B + lax.broadcasted_iota(jnp.int32, (_ROWS_B, HID), 0)
  g1_ref[...] = jnp.where(row < N, h * dinvb, 0.0)
  dinv_ref[...] = dinvb


def _tc_mid_body(a_ref, g1_ref, dinv_ref, b1_ref, w2_ref, g2_ref):
  pid = pl.program_id(0)
  dinv = dinv_ref[...]
  z1 = dinv * (a_ref[0] + a_ref[1] + g1_ref[...]) + b1_ref[...]
  z1 = jnp.maximum(z1, 0.0)
  h2 = jnp.dot(z1, w2_ref[...], preferred_element_type=_f32)
  row = pid * _ROWS_B + lax.broadcasted_iota(jnp.int32, (_ROWS_B, HID), 0)
  g2_ref[...] = jnp.where(row < N, h2 * dinv, 0.0)


def _tc_pool_body(a_ref, g2_ref, dinv_ref, b2_ref, bt_ref, out_ref):
  pid = pl.program_id(0)
  z2 = dinv_ref[...] * (a_ref[0] + a_ref[1] + g2_ref[...]) + b2_ref[...]
  bt = bt_ref[0]
  gid = lax.broadcasted_iota(jnp.int32, (NUM_GRAPHS, _ROWS_B), 0)
  onehot = (gid == jnp.broadcast_to(bt, (NUM_GRAPHS, _ROWS_B))).astype(_f32)
  contrib = jnp.dot(onehot, z2, preferred_element_type=_f32)

  @pl.when(pid == 0)
  def _():
    out_ref[...] = contrib

  @pl.when(pid > 0)
  def _():
    out_ref[...] = out_ref[...] + contrib


def _row_spec(width):
  return pl.BlockSpec((_ROWS_B, width), lambda i: (i, 0))


def _acc_spec():
  return pl.BlockSpec((NC, _ROWS_B, HID), lambda i: (0, i, 0))


def _const_spec(shape):
  nd = len(shape)
  return pl.BlockSpec(shape, lambda i: (0,) * nd)


def _tc_hist(dst_r, dst_c):
  return pl.pallas_call(
      _tc_hist_body,
      grid=(_EGRID,),
      in_specs=[pl.BlockSpec((1, 1, _EC), lambda i: (i, 0, 0)),
                pl.BlockSpec((1, _EC, 1), lambda i: (i, 0, 0))],
      out_specs=_const_spec((_HI, 128)),
      out_shape=jax.ShapeDtypeStruct((_HI, 128), _f32),
  )(dst_r, dst_c)


def _tc_prep(x, d0, d1, w1):
  return pl.pallas_call(
      _tc_prep_body,
      grid=(_GRID,),
      in_specs=[_row_spec(EMB), _row_spec(HID), _row_spec(HID),
                _const_spec((EMB, HID))],
      out_specs=[_row_spec(HID), _row_spec(HID)],
      out_shape=[jax.ShapeDtypeStruct((NP, HID), _f32),
                 jax.ShapeDtypeStruct((NP, HID), _f32)],
  )(x, d0, d1, w1)


def _tc_mid(a, g1, dinv, b1, w2):
  return pl.pallas_call(
      _tc_mid_body,
      grid=(_GRID,),
      in_specs=[_acc_spec(), _row_spec(HID), _row_spec(HID),
                _const_spec((1, HID)), _const_spec((HID, HID))],
      out_specs=_row_spec(HID),
      out_shape=jax.ShapeDtypeStruct((NP, HID), _f32),
  )(a, g1, dinv, b1, w2)


def _tc_pool(a, g2, dinv, b2, batch3):
  return pl.pallas_call(
      _tc_pool_body,
      grid=(_GRID,),
      in_specs=[_acc_spec(), _row_spec(HID), _row_spec(HID),
                _const_spec((1, HID)),
                pl.BlockSpec((1, 1, _ROWS_B), lambda i: (i, 0, 0))],
      out_specs=_const_spec((NUM_GRAPHS, HID)),
      out_shape=jax.ShapeDtypeStruct((NUM_GRAPHS, HID), _f32),
  )(a, g2, dinv, b2, batch3)


@jax.jit
def kernel(x_type, edge_index, batch, emb, W1, b1, W2, b2):
  i32 = jnp.int32
  src = edge_index[0].astype(i32)
  dst = edge_index[1].astype(i32)

  pad = N + (jnp.arange(EP - E, dtype=i32) % (NP - N))
  srcp = jnp.concatenate([src, pad])
  dstp = jnp.concatenate([dst, pad])
  src2 = srcp.reshape(NW * NB_E, EB)
  dst2 = dstp.reshape(NW * NB_E, EB)
  xt0 = jnp.concatenate(
      [x_type.astype(i32), jnp.zeros((NP - N,), i32)]).reshape(NW, NB_X, XB)
  xt = jnp.concatenate([xt0, jnp.zeros((NW, 1, XB), i32)], axis=1)
  batch3 = jnp.concatenate(
      [batch.astype(i32),
       jnp.full((NP - N,), NUM_GRAPHS, i32)]).reshape(_GRID, 1, _ROWS_B)

  zeros128 = jnp.zeros((NP, HID), _f32)
  ones128 = jnp.ones((EB, HID), _f32)

  sc_gather, conv = _make_sc_kernels()

  degp, x = sc_gather(xt, dst2, emb, ones128, zeros128)
  g1, dinv = _tc_prep(x, degp[0], degp[1], W1)

  acc1 = conv(g1, src2, dst2, zeros128)
  g2 = _tc_mid(acc1, g1, dinv, b1.reshape(1, HID), W2)

  acc2 = conv(g2, src2, dst2, zeros128)
  out = _tc_pool(acc2, g2, dinv, b2.reshape(1, HID), batch3)
  return out


# speedup vs baseline: 1.7338x; 1.0991x over previous
import jax
import jax.numpy as jnp
from jax import lax
from jax.experimental import pallas as pl
from jax.experimental.pallas import tpu as pltpu
from jax.experimental.pallas import tpu_sc as plsc

N = 10000
E = 320000
NUM_TYPES = 512
EMB = 128
HID = 128
NUM_GRAPHS = 64

NC = 2
NS = 16
NW = NC * NS
EB = 128
NB_E = 80
CB = 40
EP = NW * NB_E * EB
NP = 10240
RPT = NP // NS
XB = 40
NB_X = (NP // NW) // XB

_f32 = jnp.float32
_bf16 = jnp.bfloat16


def _worker_id():
  c = lax.axis_index("c")
  s = lax.axis_index("s")
  return s * NC + c, c, s


def _sc_deg_gather_body(xt_hbm, dst_hbm, emb_hbm, ones_hbm, zeros_hbm,
                        deg_out, x_out,
                        dst_idx, xt_idx, ones_v, xr0, xr1, sx0, sx1, acc):
  wid, c, s = _worker_id()
  pltpu.sync_copy(dst_hbm.at[pl.ds(wid * NB_E, NB_E)], dst_idx)
  pltpu.sync_copy(xt_hbm.at[wid], xt_idx)
  pltpu.sync_copy(ones_hbm, ones_v)
  pltpu.sync_copy(zeros_hbm.at[pl.ds(s * RPT, RPT)],
                  acc.at[pl.ds(s * RPT, RPT)])
  plsc.subcore_barrier()

  def gx(j, buf, sem):
    pltpu.async_copy(emb_hbm.at[xt_idx.at[j]], buf, sem)

  def wx(j, buf, sem):
    pltpu.make_async_copy(emb_hbm.at[xt_idx.at[j]], buf, sem).wait()

  def deg_burst(base):
    def deg_body(b):
      pltpu.sync_copy(ones_v, acc.at[dst_idx.at[b]], add=True)
    pl.loop(base, base + NB_E // NB_X)(deg_body)

  gx(0, xr0, sx0)

  def phase(j):
    burst = NB_E // NB_X
    wx(j, xr0, sx0)
    gx(j + 1, xr1, sx1)
    deg_burst(j * burst)
    pltpu.sync_copy(xr0, x_out.at[pl.ds(wid * (NB_X * XB) + j * XB, XB)])
    wx(j + 1, xr1, sx1)
    gx(j + 2, xr0, sx0)
    deg_burst((j + 1) * burst)
    pltpu.sync_copy(xr1, x_out.at[pl.ds(wid * (NB_X * XB) + (j + 1) * XB, XB)])

  pl.loop(0, NB_X, step=2)(phase)
  wx(NB_X, xr0, sx0)

  plsc.subcore_barrier()
  pltpu.sync_copy(acc.at[pl.ds(s * RPT, RPT)],
                  deg_out.at[c, pl.ds(s * RPT, RPT)])


def _sc_conv_body(g_hbm, src_hbm, dst_hbm, zeros_hbm,
                  acc_out,
                  src_c, dst_c, rows0, rows1, sem0, sem1, acc):
  wid, c, s = _worker_id()
  pltpu.sync_copy(zeros_hbm.at[pl.ds(s * RPT, RPT)],
                  acc.at[pl.ds(s * RPT, RPT)])
  plsc.subcore_barrier()

  def chunk(j):
    base = wid * NB_E + j * CB
    pltpu.sync_copy(src_hbm.at[pl.ds(base, CB)], src_c)
    pltpu.sync_copy(dst_hbm.at[pl.ds(base, CB)], dst_c)
    pltpu.async_copy(g_hbm.at[src_c.at[0]], rows0, sem0)

    def pair(t):
      pltpu.async_copy(g_hbm.at[src_c.at[t + 1]], rows1, sem1)
      pltpu.make_async_copy(g_hbm.at[src_c.at[t]], rows0, sem0).wait()
      pltpu.sync_copy(rows0, acc.at[dst_c.at[t]], add=True)

      @pl.when(t + 2 < CB)
      def _():
        pltpu.async_copy(g_hbm.at[src_c.at[t + 2]], rows0, sem0)
      pltpu.make_async_copy(g_hbm.at[src_c.at[t + 1]], rows1, sem1).wait()
      pltpu.sync_copy(rows1, acc.at[dst_c.at[t + 1]], add=True)

    pl.loop(0, CB, step=2)(pair)

  pl.loop(0, NB_E // CB)(chunk)

  plsc.subcore_barrier()
  pltpu.sync_copy(acc.at[pl.ds(s * RPT, RPT)],
                  acc_out.at[c, pl.ds(s * RPT, RPT)])


def _make_sc_kernels():
  mesh = plsc.VectorSubcoreMesh(core_axis_name="c", subcore_axis_name="s")
  gather = pl.kernel(
      _sc_deg_gather_body,
      out_type=(
          jax.ShapeDtypeStruct((NC, NP, HID), _f32),
          jax.ShapeDtypeStruct((NP, EMB), _f32),
      ),
      mesh=mesh,
      scratch_types=[
          pltpu.VMEM((NB_E, EB), jnp.int32),
          pltpu.VMEM((NB_X + 1, XB), jnp.int32),
          pltpu.VMEM((EB, HID), _f32),
          pltpu.VMEM((XB, EMB), _f32),
          pltpu.VMEM((XB, EMB), _f32),
          pltpu.SemaphoreType.DMA,
          pltpu.SemaphoreType.DMA,
          pltpu.VMEM_SHARED((NP, HID), _f32),
      ],
      name="gnn_sc_deg_gather",
  )
  conv = pl.kernel(
      _sc_conv_body,
      out_type=jax.ShapeDtypeStruct((NC, NP, HID), _f32),
      mesh=mesh,
      scratch_types=[
          pltpu.VMEM((CB, EB), jnp.int32),
          pltpu.VMEM((CB, EB), jnp.int32),
          pltpu.VMEM((EB, HID), _f32),
          pltpu.VMEM((EB, HID), _f32),
          pltpu.SemaphoreType.DMA,
          pltpu.SemaphoreType.DMA,
          pltpu.VMEM_SHARED((NP, HID), _f32),
      ],
      name="gnn_sc_conv",
  )
  return gather, conv


_ROWS_B = 1024
_GRID = NP // _ROWS_B
_EC = 8192
_EGRID = EP // _EC
_HI = NP // 128


def _tc_hist_body(dr_ref, dc_ref, dm_ref):
  pid = pl.program_id(0)
  hi_r = dr_ref[0] >> 7
  lo_c = dc_ref[0] & 127
  mask_hi = (lax.broadcasted_iota(jnp.int32, (_HI, _EC), 0)
             == jnp.broadcast_to(hi_r, (_HI, _EC))).astype(_bf16)
  onehot = (lax.broadcasted_iota(jnp.int32, (_EC, 128), 1)
            == jnp.broadcast_to(lo_c, (_EC, 128))).astype(_bf16)
  contrib = jnp.dot(mask_hi, onehot, preferred_element_type=_f32)

  @pl.when(pid == 0)
  def _():
    dm_ref[...] = contrib

  @pl.when(pid > 0)
  def _():
    dm_ref[...] = dm_ref[...] + contrib


def _tc_prep_body(x_ref, d0_ref, d1_ref, w1_ref, g1_ref, dinv_ref):
  pid = pl.program_id(0)
  deg = d0_ref[:, 0:1] + d1_ref[:, 0:1]
  dinv = lax.rsqrt(deg + 1.0)
  dinvb = jnp.broadcast_to(dinv, (_ROWS_B, HID))
  h = jnp.dot(x_ref[...], w1_ref[...], preferred_element_type=_f32)
  row = pid * _ROWS_B + lax.broadcasted_iota(jnp.int32, (_ROWS_B, HID), 0)
  g1_ref[...] = jnp.where(row < N, h * dinvb, 0.0)
  dinv_ref[...] = dinvb


def _tc_mid_body(a_ref, g1_ref, dinv_ref, b1_ref, w2_ref, g2_ref):
  pid = pl.program_id(0)
  dinv = dinv_ref[...]
  z1 = dinv * (a_ref[0] + a_ref[1] + g1_ref[...]) + b1_ref[...]
  z1 = jnp.maximum(z1, 0.0)
  h2 = jnp.dot(z1, w2_ref[...], preferred_element_type=_f32)
  row = pid * _ROWS_B + lax.broadcasted_iota(jnp.int32, (_ROWS_B, HID), 0)
  g2_ref[...] = jnp.where(row < N, h2 * dinv, 0.0)


def _tc_pool_body(a_ref, g2_ref, dinv_ref, b2_ref, bt_ref, out_ref):
  pid = pl.program_id(0)
  z2 = dinv_ref[...] * (a_ref[0] + a_ref[1] + g2_ref[...]) + b2_ref[...]
  bt = bt_ref[0]
  gid = lax.broadcasted_iota(jnp.int32, (NUM_GRAPHS, _ROWS_B), 0)
  onehot = (gid == jnp.broadcast_to(bt, (NUM_GRAPHS, _ROWS_B))).astype(_f32)
  contrib = jnp.dot(onehot, z2, preferred_element_type=_f32)

  @pl.when(pid == 0)
  def _():
    out_ref[...] = contrib

  @pl.when(pid > 0)
  def _():
    out_ref[...] = out_ref[...] + contrib


def _row_spec(width):
  return pl.BlockSpec((_ROWS_B, width), lambda i: (i, 0))


def _acc_spec():
  return pl.BlockSpec((NC, _ROWS_B, HID), lambda i: (0, i, 0))


def _const_spec(shape):
  nd = len(shape)
  return pl.BlockSpec(shape, lambda i: (0,) * nd)


def _tc_hist(dst_r, dst_c):
  return pl.pallas_call(
      _tc_hist_body,
      grid=(_EGRID,),
      in_specs=[pl.BlockSpec((1, 1, _EC), lambda i: (i, 0, 0)),
                pl.BlockSpec((1, _EC, 1), lambda i: (i, 0, 0))],
      out_specs=_const_spec((_HI, 128)),
      out_shape=jax.ShapeDtypeStruct((_HI, 128), _f32),
  )(dst_r, dst_c)


def _tc_prep(x, d0, d1, w1):
  return pl.pallas_call(
      _tc_prep_body,
      grid=(_GRID,),
      in_specs=[_row_spec(EMB), _row_spec(HID), _row_spec(HID),
                _const_spec((EMB, HID))],
      out_specs=[_row_spec(HID), _row_spec(HID)],
      out_shape=[jax.ShapeDtypeStruct((NP, HID), _f32),
                 jax.ShapeDtypeStruct((NP, HID), _f32)],
  )(x, d0, d1, w1)


def _tc_mid(a, g1, dinv, b1, w2):
  return pl.pallas_call(
      _tc_mid_body,
      grid=(_GRID,),
      in_specs=[_acc_spec(), _row_spec(HID), _row_spec(HID),
                _const_spec((1, HID)), _const_spec((HID, HID))],
      out_specs=_row_spec(HID),
      out_shape=jax.ShapeDtypeStruct((NP, HID), _f32),
  )(a, g1, dinv, b1, w2)


def _tc_pool(a, g2, dinv, b2, batch3):
  return pl.pallas_call(
      _tc_pool_body,
      grid=(_GRID,),
      in_specs=[_acc_spec(), _row_spec(HID), _row_spec(HID),
                _const_spec((1, HID)),
                pl.BlockSpec((1, 1, _ROWS_B), lambda i: (i, 0, 0))],
      out_specs=_const_spec((NUM_GRAPHS, HID)),
      out_shape=jax.ShapeDtypeStruct((NUM_GRAPHS, HID), _f32),
  )(a, g2, dinv, b2, batch3)


@jax.jit
def kernel(x_type, edge_index, batch, emb, W1, b1, W2, b2):
  i32 = jnp.int32
  src = edge_index[0].astype(i32)
  dst = edge_index[1].astype(i32)

  pad = N + (jnp.arange(EP - E, dtype=i32) % (NP - N))
  srcp = jnp.concatenate([src, pad])
  dstp = jnp.concatenate([dst, pad])
  src2 = srcp.reshape(NW * NB_E, EB)
  dst2 = dstp.reshape(NW * NB_E, EB)
  xt0 = jnp.concatenate(
      [x_type.astype(i32), jnp.zeros((NP - N,), i32)]).reshape(NW, NB_X, XB)
  xt = jnp.concatenate([xt0, jnp.zeros((NW, 1, XB), i32)], axis=1)
  batch3 = jnp.concatenate(
      [batch.astype(i32),
       jnp.full((NP - N,), NUM_GRAPHS, i32)]).reshape(_GRID, 1, _ROWS_B)

  zeros128 = jnp.zeros((NP, HID), _f32)
  ones128 = jnp.ones((EB, HID), _f32)

  sc_gather, conv = _make_sc_kernels()

  degp, x = sc_gather(xt, dst2, emb, ones128, zeros128)
  g1, dinv = _tc_prep(x, degp[0], degp[1], W1)

  acc1 = conv(g1, src2, dst2, zeros128)
  g2 = _tc_mid(acc1, g1, dinv, b1.reshape(1, HID), W2)

  acc2 = conv(g2, src2, dst2, zeros128)
  out = _tc_pool(acc2, g2, dinv, b2.reshape(1, HID), batch3)
  return out


# speedup vs baseline: 1.7359x; 1.0012x over previous
import jax
import jax.numpy as jnp
from jax import lax
from jax.experimental import pallas as pl
from jax.experimental.pallas import tpu as pltpu
from jax.experimental.pallas import tpu_sc as plsc

N = 10000
E = 320000
NUM_TYPES = 512
EMB = 128
HID = 128
NUM_GRAPHS = 64

NC = 2
NS = 16
NW = NC * NS
EB = 128
NB_E = 80
CB = 40
EP = NW * NB_E * EB
NP = 10240
RPT = NP // NS
XB = 40
NB_X = (NP // NW) // XB

_f32 = jnp.float32


def _worker_id():
  c = lax.axis_index("c")
  s = lax.axis_index("s")
  return s * NC + c, c, s


def _sc_deg_gather_body(xt_hbm, dst_hbm, emb_hbm, ones_hbm, zeros_hbm,
                        deg_out, x_out,
                        dst_idx, xt_idx, ones_v, xr0, xr1, sx0, sx1, acc):
  wid, c, s = _worker_id()
  pltpu.sync_copy(dst_hbm.at[pl.ds(wid * NB_E, NB_E)], dst_idx)
  pltpu.sync_copy(xt_hbm.at[wid], xt_idx)
  pltpu.sync_copy(ones_hbm, ones_v)
  pltpu.sync_copy(zeros_hbm.at[pl.ds(s * RPT, RPT)],
                  acc.at[pl.ds(s * RPT, RPT)])
  plsc.subcore_barrier()

  def gx(j, buf, sem):
    pltpu.async_copy(emb_hbm.at[xt_idx.at[j]], buf, sem)

  def wx(j, buf, sem):
    pltpu.make_async_copy(emb_hbm.at[xt_idx.at[j]], buf, sem).wait()

  def deg_burst(base):
    def deg_body(b):
      pltpu.sync_copy(ones_v, acc.at[dst_idx.at[b]], add=True)
    pl.loop(base, base + NB_E // NB_X)(deg_body)

  gx(0, xr0, sx0)

  def phase(j):
    burst = NB_E // NB_X
    wx(j, xr0, sx0)
    gx(j + 1, xr1, sx1)
    deg_burst(j * burst)
    pltpu.sync_copy(xr0, x_out.at[pl.ds(wid * (NB_X * XB) + j * XB, XB)])
    wx(j + 1, xr1, sx1)
    gx(j + 2, xr0, sx0)
    deg_burst((j + 1) * burst)
    pltpu.sync_copy(xr1, x_out.at[pl.ds(wid * (NB_X * XB) + (j + 1) * XB, XB)])

  pl.loop(0, NB_X, step=2)(phase)
  wx(NB_X, xr0, sx0)

  plsc.subcore_barrier()
  pltpu.sync_copy(acc.at[pl.ds(s * RPT, RPT)],
                  deg_out.at[c, pl.ds(s * RPT, RPT)])


def _sc_conv_body(g_hbm, src_hbm, dst_hbm, zeros_hbm,
                  acc_out,
                  src_c, dst_c, rows0, rows1, sem0, sem1, acc):
  wid, c, s = _worker_id()
  pltpu.sync_copy(zeros_hbm.at[pl.ds(s * RPT, RPT)],
                  acc.at[pl.ds(s * RPT, RPT)])
  plsc.subcore_barrier()

  def chunk(j):
    base = wid * NB_E + j * CB
    pltpu.sync_copy(src_hbm.at[pl.ds(base, CB)], src_c)
    pltpu.sync_copy(dst_hbm.at[pl.ds(base, CB)], dst_c)
    pltpu.async_copy(g_hbm.at[src_c.at[0]], rows0, sem0)

    def pair(t):
      pltpu.async_copy(g_hbm.at[src_c.at[t + 1]], rows1, sem1)
      pltpu.make_async_copy(g_hbm.at[src_c.at[t]], rows0, sem0).wait()
      pltpu.sync_copy(rows0, acc.at[dst_c.at[t]], add=True)

      @pl.when(t + 2 < CB)
      def _():
        pltpu.async_copy(g_hbm.at[src_c.at[t + 2]], rows0, sem0)
      pltpu.make_async_copy(g_hbm.at[src_c.at[t + 1]], rows1, sem1).wait()
      pltpu.sync_copy(rows1, acc.at[dst_c.at[t + 1]], add=True)

    pl.loop(0, CB, step=2)(pair)

  pl.loop(0, NB_E // CB)(chunk)

  plsc.subcore_barrier()
  pltpu.sync_copy(acc.at[pl.ds(s * RPT, RPT)],
                  acc_out.at[c, pl.ds(s * RPT, RPT)])


def _make_sc_kernels():
  mesh = plsc.VectorSubcoreMesh(core_axis_name="c", subcore_axis_name="s")
  gather = pl.kernel(
      _sc_deg_gather_body,
      out_type=(
          jax.ShapeDtypeStruct((NC, NP, HID), _f32),
          jax.ShapeDtypeStruct((NP, EMB), _f32),
      ),
      mesh=mesh,
      scratch_types=[
          pltpu.VMEM((NB_E, EB), jnp.int32),
          pltpu.VMEM((NB_X + 1, XB), jnp.int32),
          pltpu.VMEM((EB, HID), _f32),
          pltpu.VMEM((XB, EMB), _f32),
          pltpu.VMEM((XB, EMB), _f32),
          pltpu.SemaphoreType.DMA,
          pltpu.SemaphoreType.DMA,
          pltpu.VMEM_SHARED((NP, HID), _f32),
      ],
      name="gnn_sc_deg_gather",
  )
  conv = pl.kernel(
      _sc_conv_body,
      out_type=jax.ShapeDtypeStruct((NC, NP, HID), _f32),
      mesh=mesh,
      scratch_types=[
          pltpu.VMEM((CB, EB), jnp.int32),
          pltpu.VMEM((CB, EB), jnp.int32),
          pltpu.VMEM((EB, HID), _f32),
          pltpu.VMEM((EB, HID), _f32),
          pltpu.SemaphoreType.DMA,
          pltpu.SemaphoreType.DMA,
          pltpu.VMEM_SHARED((NP, HID), _f32),
      ],
      name="gnn_sc_conv",
  )
  return gather, conv


_ROWS_B = 1024
_GRID = NP // _ROWS_B

def _tc_prep_body(x_ref, d0_ref, d1_ref, w1_ref, g1_ref, dinv_ref):
  pid = pl.program_id(0)
  deg = d0_ref[:, 0:1] + d1_ref[:, 0:1]
  dinv = lax.rsqrt(deg + 1.0)
  dinvb = jnp.broadcast_to(dinv, (_ROWS_B, HID))
  h = jnp.dot(x_ref[...], w1_ref[...], preferred_element_type=_f32)
  row = pid * _ROWS_B + lax.broadcasted_iota(jnp.int32, (_ROWS_B, HID), 0)
  g1_ref[...] = jnp.where(row < N, h * dinvb, 0.0)
  dinv_ref[...] = dinvb


def _tc_mid_body(a_ref, g1_ref, dinv_ref, b1_ref, w2_ref, g2_ref):
  pid = pl.program_id(0)
  dinv = dinv_ref[...]
  z1 = dinv * (a_ref[0] + a_ref[1] + g1_ref[...]) + b1_ref[...]
  z1 = jnp.maximum(z1, 0.0)
  h2 = jnp.dot(z1, w2_ref[...], preferred_element_type=_f32)
  row = pid * _ROWS_B + lax.broadcasted_iota(jnp.int32, (_ROWS_B, HID), 0)
  g2_ref[...] = jnp.where(row < N, h2 * dinv, 0.0)


def _tc_pool_body(a_ref, g2_ref, dinv_ref, b2_ref, bt_ref, out_ref):
  pid = pl.program_id(0)
  z2 = dinv_ref[...] * (a_ref[0] + a_ref[1] + g2_ref[...]) + b2_ref[...]
  bt = bt_ref[0]
  gid = lax.broadcasted_iota(jnp.int32, (NUM_GRAPHS, _ROWS_B), 0)
  onehot = (gid == jnp.broadcast_to(bt, (NUM_GRAPHS, _ROWS_B))).astype(_f32)
  contrib = jnp.dot(onehot, z2, preferred_element_type=_f32)

  @pl.when(pid == 0)
  def _():
    out_ref[...] = contrib

  @pl.when(pid > 0)
  def _():
    out_ref[...] = out_ref[...] + contrib


def _row_spec(width):
  return pl.BlockSpec((_ROWS_B, width), lambda i: (i, 0))


def _acc_spec():
  return pl.BlockSpec((NC, _ROWS_B, HID), lambda i: (0, i, 0))


def _const_spec(shape):
  nd = len(shape)
  return pl.BlockSpec(shape, lambda i: (0,) * nd)


def _tc_prep(x, d0, d1, w1):
  return pl.pallas_call(
      _tc_prep_body,
      grid=(_GRID,),
      in_specs=[_row_spec(EMB), _row_spec(HID), _row_spec(HID),
                _const_spec((EMB, HID))],
      out_specs=[_row_spec(HID), _row_spec(HID)],
      out_shape=[jax.ShapeDtypeStruct((NP, HID), _f32),
                 jax.ShapeDtypeStruct((NP, HID), _f32)],
  )(x, d0, d1, w1)


def _tc_mid(a, g1, dinv, b1, w2):
  return pl.pallas_call(
      _tc_mid_body,
      grid=(_GRID,),
      in_specs=[_acc_spec(), _row_spec(HID), _row_spec(HID),
                _const_spec((1, HID)), _const_spec((HID, HID))],
      out_specs=_row_spec(HID),
      out_shape=jax.ShapeDtypeStruct((NP, HID), _f32),
  )(a, g1, dinv, b1, w2)


def _tc_pool(a, g2, dinv, b2, batch3):
  return pl.pallas_call(
      _tc_pool_body,
      grid=(_GRID,),
      in_specs=[_acc_spec(), _row_spec(HID), _row_spec(HID),
                _const_spec((1, HID)),
                pl.BlockSpec((1, 1, _ROWS_B), lambda i: (i, 0, 0))],
      out_specs=_const_spec((NUM_GRAPHS, HID)),
      out_shape=jax.ShapeDtypeStruct((NUM_GRAPHS, HID), _f32),
  )(a, g2, dinv, b2, batch3)


@jax.jit
def kernel(x_type, edge_index, batch, emb, W1, b1, W2, b2):
  i32 = jnp.int32
  src = edge_index[0].astype(i32)
  dst = edge_index[1].astype(i32)

  pad = N + (jnp.arange(EP - E, dtype=i32) % (NP - N))
  srcp = jnp.concatenate([src, pad])
  dstp = jnp.concatenate([dst, pad])
  src2 = srcp.reshape(NW * NB_E, EB)
  dst2 = dstp.reshape(NW * NB_E, EB)
  xt0 = jnp.concatenate(
      [x_type.astype(i32), jnp.zeros((NP - N,), i32)]).reshape(NW, NB_X, XB)
  xt = jnp.concatenate([xt0, jnp.zeros((NW, 1, XB), i32)], axis=1)
  batch3 = jnp.concatenate(
      [batch.astype(i32),
       jnp.full((NP - N,), NUM_GRAPHS, i32)]).reshape(_GRID, 1, _ROWS_B)

  zeros128 = jnp.zeros((NP, HID), _f32)
  ones128 = jnp.ones((EB, HID), _f32)

  sc_gather, conv = _make_sc_kernels()

  degp, x = sc_gather(xt, dst2, emb, ones128, zeros128)
  g1, dinv = _tc_prep(x, degp[0], degp[1], W1)

  acc1 = conv(g1, src2, dst2, zeros128)
  g2 = _tc_mid(acc1, g1, dinv, b1.reshape(1, HID), W2)

  acc2 = conv(g2, src2, dst2, zeros128)
  out = _tc_pool(acc2, g2, dinv, b2.reshape(1, HID), batch3)
  return out
